# baseline (speedup 1.0000x reference)
import functools

import jax
import jax.numpy as jnp
from jax import lax
from jax.experimental import pallas as pl
from jax.experimental.pallas import tpu as pltpu
from jax.experimental.pallas import tpu_sc as plsc

KNN = 3
NBLK = 1024
SC_G = 32


def _dotg(a, b, dims):
    return jax.lax.dot_general(a, b, (dims, ((), ())),
                               preferred_element_type=jnp.float32)


def _k1_body(uc_ref, kcs_ref, idx_ref, w_ref, *, m_per_b):
    uc = uc_ref[0]
    kcs = kcs_ref[0]
    n, m = uc.shape[1], kcs.shape[1]

    su = jnp.sum(uc * uc, axis=0, keepdims=True).T
    sv = 0.25 * jnp.sum(kcs * kcs, axis=0, keepdims=True)
    ndot2 = _dotg(uc, kcs, ((0,), (0,)))
    d = (su + sv) + ndot2

    v1 = jnp.min(d, axis=1, keepdims=True)
    d2 = jnp.where(d == v1, jnp.inf, d)
    v2 = jnp.min(d2, axis=1, keepdims=True)
    d3 = jnp.where(d2 == v2, jnp.inf, d2)
    v3 = jnp.min(d3, axis=1, keepdims=True)
    dists = [v1, v2, v3]

    wk = [1.0 / (jnp.maximum(v, 0.0) + 1e-8) for v in dists]
    ws = (wk[0] + wk[1]) + wk[2]
    wn = [w / ws for w in wk]

    iota = jax.lax.broadcasted_iota(jnp.int32, (n, m), 1)
    big = jnp.int32(m)
    off = pl.program_id(0) * m_per_b
    i1 = jnp.min(jnp.where(d == v1, iota, big), axis=1, keepdims=True)
    i2 = jnp.min(jnp.where(d2 == v2, iota, big), axis=1, keepdims=True)
    i3 = jnp.min(jnp.where(d3 == v3, iota, big), axis=1, keepdims=True)
    idx_ref[0] = jnp.concatenate([i1, i2, i3], axis=1) + off
    w_ref[0] = jnp.concatenate(wn, axis=1)


def _k2_body(inter_ref, uf_ref, w1ta_ref, w1tb_ref, b1_ref, y1_ref, s1_ref):
    y1 = (jnp.dot(inter_ref[0], w1ta_ref[...],
                  preferred_element_type=jnp.float32)
          + _dotg(uf_ref[0], w1tb_ref[...], ((0,), (0,)))
          + b1_ref[...])
    y1_ref[0] = y1
    s1_ref[0, 0, 0:1, :] = jnp.sum(y1, axis=0, keepdims=True)
    s1_ref[0, 0, 1:2, :] = jnp.sum(y1 * y1, axis=0, keepdims=True)


def _bn_relu(y, stats, g_ref, be_ref, count):
    ssum = jnp.sum(stats[:, :, 0, :], axis=(0, 1), keepdims=False)
    ssq = jnp.sum(stats[:, :, 1, :], axis=(0, 1), keepdims=False)
    mu = (ssum / count)[None, :]
    var = (ssq / count)[None, :] - mu * mu
    rstd = jax.lax.rsqrt(var + 1e-5)
    z = (y - mu) * rstd * g_ref[...] + be_ref[...]
    return jnp.maximum(z, 0.0)


def _k3_body(y1_ref, s1_ref, g1_ref, be1_ref, w2t_ref, b2_ref, y2_ref, s2_ref,
             *, count):
    z = _bn_relu(y1_ref[0], s1_ref[...], g1_ref, be1_ref, count)
    y2 = jnp.dot(z, w2t_ref[...], preferred_element_type=jnp.float32) + b2_ref[...]
    y2_ref[0] = y2
    s2_ref[0, 0, 0:1, :] = jnp.sum(y2, axis=0, keepdims=True)
    s2_ref[0, 0, 1:2, :] = jnp.sum(y2 * y2, axis=0, keepdims=True)


def _k4_body(y2_ref, s2_ref, g2_ref, be2_ref, out_ref, *, count):
    z = _bn_relu(y2_ref[0], s2_ref[...], g2_ref, be2_ref, count)
    out_ref[0] = z.T


def _sc_gather_combine(kf_flat, idxf, wf, P, C2):
    info = pltpu.get_tpu_info().sparse_core
    NC, NS = info.num_cores, info.num_subcores
    NW = NC * NS
    pw = P // NW
    G = SC_G
    mesh = plsc.VectorSubcoreMesh(core_axis_name="c", subcore_axis_name="s")
    f32 = jnp.float32

    @functools.partial(
        pl.kernel, mesh=mesh,
        out_type=jax.ShapeDtypeStruct((P, C2), f32),
        scratch_types=[
            pltpu.VMEM((G,), jnp.int32),
            pltpu.VMEM((G,), jnp.int32),
            pltpu.VMEM((G,), jnp.int32),
            pltpu.VMEM((G, C2), f32),
            pltpu.VMEM((G, C2), f32),
            pltpu.VMEM((G, C2), f32),
            pltpu.VMEM((G, C2), f32),
            pltpu.VMEM((3 * G + 16,), f32),
            pltpu.SemaphoreType.DMA,
        ],
    )
    def sc_k(kf_hbm, idx_hbm, w_hbm, out_hbm,
             i0, i1, i2, r0, r1, r2, ro, wv, sem):
        wid = lax.axis_index("s") * NC + lax.axis_index("c")
        base = wid * pw

        @pl.loop(0, pw, step=G)
        def _(off):
            b0 = base + off
            pltpu.sync_copy(idx_hbm.at[pl.ds(0 * P + b0, G)], i0)
            pltpu.sync_copy(idx_hbm.at[pl.ds(1 * P + b0, G)], i1)
            pltpu.sync_copy(idx_hbm.at[pl.ds(2 * P + b0, G)], i2)
            pltpu.sync_copy(w_hbm.at[pl.ds(0 * P + b0, G)],
                            wv.at[pl.ds(0, G)])
            pltpu.sync_copy(w_hbm.at[pl.ds(1 * P + b0, G)],
                            wv.at[pl.ds(G, G)])
            pltpu.sync_copy(w_hbm.at[pl.ds(2 * P + b0, G)],
                            wv.at[pl.ds(2 * G, G)])
            pltpu.async_copy(kf_hbm.at[i0], r0, sem).wait()
            pltpu.async_copy(kf_hbm.at[i1], r1, sem).wait()
            pltpu.async_copy(kf_hbm.at[i2], r2, sem).wait()

            @pl.loop(0, G)
            def _(r):
                w0 = wv[pl.ds(r, 16)][0]
                w1 = wv[pl.ds(G + r, 16)][0]
                w2 = wv[pl.ds(2 * G + r, 16)][0]

                @pl.loop(0, C2, step=16)
                def _(c):
                    slc = (r, pl.ds(c, 16))
                    ro.at[*slc][...] = (w0 * r0.at[*slc][...]
                                        + w1 * r1.at[*slc][...]
                                        + w2 * r2.at[*slc][...])

            pltpu.sync_copy(ro, out_hbm.at[pl.ds(b0, G)])

    return sc_k(kf_flat, idxf, wf)


def kernel(unknown_coords, known_coords, unknown_feats, known_feats,
           W1, b1, g1, be1, W2, b2, g2, be2):
    B, _, n = unknown_coords.shape
    m = known_coords.shape[2]
    C1 = unknown_feats.shape[1]
    C2 = known_feats.shape[1]
    CO1 = W1.shape[0]
    CO2 = W2.shape[0]
    nb = n // NBLK
    P = B * n
    count = float(B * n)

    w1ta = W1[:, :C2].T
    w1tb = W1[:, C2:].T
    w2t = W2.T
    b1r = b1.reshape(1, CO1)
    g1r = g1.reshape(1, CO1)
    be1r = be1.reshape(1, CO1)
    b2r = b2.reshape(1, CO2)
    g2r = g2.reshape(1, CO2)
    be2r = be2.reshape(1, CO2)

    grid = (B, nb)
    f32 = jnp.float32

    idx, w = pl.pallas_call(
        functools.partial(_k1_body, m_per_b=m),
        grid=grid,
        in_specs=[
            pl.BlockSpec((1, 3, NBLK), lambda b, i: (b, 0, i)),
            pl.BlockSpec((1, 3, m), lambda b, i: (b, 0, 0)),
        ],
        out_specs=[
            pl.BlockSpec((1, NBLK, KNN), lambda b, i: (b, i, 0)),
            pl.BlockSpec((1, NBLK, KNN), lambda b, i: (b, i, 0)),
        ],
        out_shape=[
            jax.ShapeDtypeStruct((B, n, KNN), jnp.int32),
            jax.ShapeDtypeStruct((B, n, KNN), f32),
        ],
    )(unknown_coords, -2.0 * known_coords)

    kf_flat = known_feats.transpose(0, 2, 1).reshape(P // n * m, C2)
    idxf = idx.transpose(2, 0, 1).reshape(KNN * P)
    wf = w.transpose(2, 0, 1).reshape(KNN * P)
    inter = _sc_gather_combine(kf_flat, idxf, wf, P, C2).reshape(B, n, C2)

    y1, s1 = pl.pallas_call(
        _k2_body,
        grid=grid,
        in_specs=[
            pl.BlockSpec((1, NBLK, C2), lambda b, i: (b, i, 0)),
            pl.BlockSpec((1, C1, NBLK), lambda b, i: (b, 0, i)),
            pl.BlockSpec((C2, CO1), lambda b, i: (0, 0)),
            pl.BlockSpec((C1, CO1), lambda b, i: (0, 0)),
            pl.BlockSpec((1, CO1), lambda b, i: (0, 0)),
        ],
        out_specs=[
            pl.BlockSpec((1, NBLK, CO1), lambda b, i: (b, i, 0)),
            pl.BlockSpec((1, 1, 2, CO1), lambda b, i: (b, i, 0, 0)),
        ],
        out_shape=[
            jax.ShapeDtypeStruct((B, n, CO1), f32),
            jax.ShapeDtypeStruct((B, nb, 2, CO1), f32),
        ],
    )(inter, unknown_feats, w1ta, w1tb, b1r)

    y2, s2 = pl.pallas_call(
        functools.partial(_k3_body, count=count),
        grid=grid,
        in_specs=[
            pl.BlockSpec((1, NBLK, CO1), lambda b, i: (b, i, 0)),
            pl.BlockSpec((B, nb, 2, CO1), lambda b, i: (0, 0, 0, 0)),
            pl.BlockSpec((1, CO1), lambda b, i: (0, 0)),
            pl.BlockSpec((1, CO1), lambda b, i: (0, 0)),
            pl.BlockSpec((CO1, CO2), lambda b, i: (0, 0)),
            pl.BlockSpec((1, CO2), lambda b, i: (0, 0)),
        ],
        out_specs=[
            pl.BlockSpec((1, NBLK, CO2), lambda b, i: (b, i, 0)),
            pl.BlockSpec((1, 1, 2, CO2), lambda b, i: (b, i, 0, 0)),
        ],
        out_shape=[
            jax.ShapeDtypeStruct((B, n, CO2), f32),
            jax.ShapeDtypeStruct((B, nb, 2, CO2), f32),
        ],
    )(y1, s1, g1r, be1r, w2t, b2r)

    out = pl.pallas_call(
        functools.partial(_k4_body, count=count),
        grid=grid,
        in_specs=[
            pl.BlockSpec((1, NBLK, CO2), lambda b, i: (b, i, 0)),
            pl.BlockSpec((B, nb, 2, CO2), lambda b, i: (0, 0, 0, 0)),
            pl.BlockSpec((1, CO2), lambda b, i: (0, 0)),
            pl.BlockSpec((1, CO2), lambda b, i: (0, 0)),
        ],
        out_specs=pl.BlockSpec((1, CO2, NBLK), lambda b, i: (b, 0, i)),
        out_shape=jax.ShapeDtypeStruct((B, CO2, n), f32),
    )(y2, s2, g2r, be2r)
    return out


# speedup vs baseline: 2.0411x; 2.0411x over previous
import jax
import jax.numpy as jnp
from jax.experimental import pallas as pl

KNN = 3
NBLK = 1024


def _dotg(a, b, dims):
    return jax.lax.dot_general(a, b, (dims, ((), ())),
                               preferred_element_type=jnp.float32)


def _k1_body(uc_ref, kcs_ref, uf_ref, kf_ref, w1ta_ref, w1tb_ref, b1_ref,
             y1_ref, s1_ref):
    uc = uc_ref[0]
    kcs = kcs_ref[0]
    n, m = uc.shape[1], kcs.shape[1]

    su = jnp.sum(uc * uc, axis=0, keepdims=True).T
    sv = 0.25 * jnp.sum(kcs * kcs, axis=0, keepdims=True)
    ndot2 = _dotg(uc, kcs, ((0,), (0,)))
    d = (su + sv) + ndot2

    v1 = jnp.min(d, axis=1, keepdims=True)
    d2 = jnp.where(d == v1, jnp.inf, d)
    v2 = jnp.min(d2, axis=1, keepdims=True)
    d3 = jnp.where(d2 == v2, jnp.inf, d2)
    v3 = jnp.min(d3, axis=1, keepdims=True)
    dists = [v1, v2, v3]

    wk = [1.0 / (jnp.maximum(v, 0.0) + 1e-8) for v in dists]
    ws = (wk[0] + wk[1]) + wk[2]
    wn = [w / ws for w in wk]

    s_mat = jnp.where(d == v1, wn[0],
                      jnp.where(d == v2, wn[1],
                                jnp.where(d3 == v3, wn[2], 0.0))
                      ).astype(jnp.bfloat16)
    inter = _dotg(s_mat, kf_ref[0], ((1,), (1,)))

    y1 = (jnp.dot(inter, w1ta_ref[...], preferred_element_type=jnp.float32)
          + _dotg(uf_ref[0], w1tb_ref[...], ((0,), (0,)))
          + b1_ref[...])
    y1_ref[0] = y1
    s1_ref[0, 0, 0:1, :] = jnp.sum(y1, axis=0, keepdims=True)
    s1_ref[0, 0, 1:2, :] = jnp.sum(y1 * y1, axis=0, keepdims=True)


def _bn_relu(y, stats, g_ref, be_ref, count):
    ssum = jnp.sum(stats[:, :, 0, :], axis=(0, 1), keepdims=False)
    ssq = jnp.sum(stats[:, :, 1, :], axis=(0, 1), keepdims=False)
    mu = (ssum / count)[None, :]
    var = (ssq / count)[None, :] - mu * mu
    rstd = jax.lax.rsqrt(var + 1e-5)
    z = (y - mu) * rstd * g_ref[...] + be_ref[...]
    return jnp.maximum(z, 0.0)


def _k2_body(y1_ref, s1_ref, g1_ref, be1_ref, w2t_ref, b2_ref, y2_ref, s2_ref,
             *, count):
    z = _bn_relu(y1_ref[0], s1_ref[...], g1_ref, be1_ref, count)
    y2 = jnp.dot(z, w2t_ref[...], preferred_element_type=jnp.float32) + b2_ref[...]
    y2_ref[0] = y2
    s2_ref[0, 0, 0:1, :] = jnp.sum(y2, axis=0, keepdims=True)
    s2_ref[0, 0, 1:2, :] = jnp.sum(y2 * y2, axis=0, keepdims=True)


def _k3_body(y2_ref, s2_ref, g2_ref, be2_ref, out_ref, *, count):
    z = _bn_relu(y2_ref[0], s2_ref[...], g2_ref, be2_ref, count)
    out_ref[0] = z.T


def kernel(unknown_coords, known_coords, unknown_feats, known_feats,
           W1, b1, g1, be1, W2, b2, g2, be2):
    B, _, n = unknown_coords.shape
    m = known_coords.shape[2]
    C1 = unknown_feats.shape[1]
    C2 = known_feats.shape[1]
    CO1 = W1.shape[0]
    CO2 = W2.shape[0]
    nb = n // NBLK
    count = float(B * n)

    w1ta = W1[:, :C2].T
    w1tb = W1[:, C2:].T
    w2t = W2.T
    b1r = b1.reshape(1, CO1)
    g1r = g1.reshape(1, CO1)
    be1r = be1.reshape(1, CO1)
    b2r = b2.reshape(1, CO2)
    g2r = g2.reshape(1, CO2)
    be2r = be2.reshape(1, CO2)

    grid = (B, nb)
    f32 = jnp.float32

    y1, s1 = pl.pallas_call(
        _k1_body,
        grid=grid,
        in_specs=[
            pl.BlockSpec((1, 3, NBLK), lambda b, i: (b, 0, i)),
            pl.BlockSpec((1, 3, m), lambda b, i: (b, 0, 0)),
            pl.BlockSpec((1, C1, NBLK), lambda b, i: (b, 0, i)),
            pl.BlockSpec((1, C2, m), lambda b, i: (b, 0, 0)),
            pl.BlockSpec((C2, CO1), lambda b, i: (0, 0)),
            pl.BlockSpec((C1, CO1), lambda b, i: (0, 0)),
            pl.BlockSpec((1, CO1), lambda b, i: (0, 0)),
        ],
        out_specs=[
            pl.BlockSpec((1, NBLK, CO1), lambda b, i: (b, i, 0)),
            pl.BlockSpec((1, 1, 2, CO1), lambda b, i: (b, i, 0, 0)),
        ],
        out_shape=[
            jax.ShapeDtypeStruct((B, n, CO1), f32),
            jax.ShapeDtypeStruct((B, nb, 2, CO1), f32),
        ],
    )(unknown_coords, -2.0 * known_coords, unknown_feats,
      known_feats.astype(jnp.bfloat16), w1ta, w1tb, b1r)

    from functools import partial
    y2, s2 = pl.pallas_call(
        partial(_k2_body, count=count),
        grid=grid,
        in_specs=[
            pl.BlockSpec((1, NBLK, CO1), lambda b, i: (b, i, 0)),
            pl.BlockSpec((B, nb, 2, CO1), lambda b, i: (0, 0, 0, 0)),
            pl.BlockSpec((1, CO1), lambda b, i: (0, 0)),
            pl.BlockSpec((1, CO1), lambda b, i: (0, 0)),
            pl.BlockSpec((CO1, CO2), lambda b, i: (0, 0)),
            pl.BlockSpec((1, CO2), lambda b, i: (0, 0)),
        ],
        out_specs=[
            pl.BlockSpec((1, NBLK, CO2), lambda b, i: (b, i, 0)),
            pl.BlockSpec((1, 1, 2, CO2), lambda b, i: (b, i, 0, 0)),
        ],
        out_shape=[
            jax.ShapeDtypeStruct((B, n, CO2), f32),
            jax.ShapeDtypeStruct((B, nb, 2, CO2), f32),
        ],
    )(y1, s1, g1r, be1r, w2t, b2r)

    out = pl.pallas_call(
        partial(_k3_body, count=count),
        grid=grid,
        in_specs=[
            pl.BlockSpec((1, NBLK, CO2), lambda b, i: (b, i, 0)),
            pl.BlockSpec((B, nb, 2, CO2), lambda b, i: (0, 0, 0, 0)),
            pl.BlockSpec((1, CO2), lambda b, i: (0, 0)),
            pl.BlockSpec((1, CO2), lambda b, i: (0, 0)),
        ],
        out_specs=pl.BlockSpec((1, CO2, NBLK), lambda b, i: (b, 0, i)),
        out_shape=jax.ShapeDtypeStruct((B, CO2, n), f32),
    )(y2, s2, g2r, be2r)
    return out


# speedup vs baseline: 2.2211x; 1.0882x over previous
import jax
import jax.numpy as jnp
from jax.experimental import pallas as pl

KNN = 3
NBLK = 1024


def _dotg(a, b, dims):
    return jax.lax.dot_general(a, b, (dims, ((), ())),
                               preferred_element_type=jnp.float32)


def _k1_body(uc_ref, kcs_ref, uf_ref, kf_ref, w1ta_ref, w1tb_ref, b1_ref,
             y1_ref, s1_ref):
    uc = uc_ref[0]
    kcs = kcs_ref[0]
    n, m = uc.shape[1], kcs.shape[1]

    su = jnp.sum(uc * uc, axis=0, keepdims=True).T
    sv = 0.25 * jnp.sum(kcs * kcs, axis=0, keepdims=True)
    ndot2 = _dotg(uc, kcs, ((0,), (0,)))
    d = (su + sv) + ndot2

    v1 = jnp.min(d, axis=1, keepdims=True)
    d2 = jnp.where(d == v1, jnp.inf, d)
    v2 = jnp.min(d2, axis=1, keepdims=True)
    d3 = jnp.where(d2 == v2, jnp.inf, d2)
    v3 = jnp.min(d3, axis=1, keepdims=True)
    dists = [v1, v2, v3]

    wk = [1.0 / (jnp.maximum(v, 0.0) + 1e-8) for v in dists]
    ws = (wk[0] + wk[1]) + wk[2]
    wn = [w / ws for w in wk]

    s_mat = jnp.where(d == v1, wn[0],
                      jnp.where(d == v2, wn[1],
                                jnp.where(d3 == v3, wn[2], 0.0)))
    inter = _dotg(s_mat, kf_ref[0], ((1,), (1,)))

    y1 = (jnp.dot(inter, w1ta_ref[...], preferred_element_type=jnp.float32)
          + _dotg(uf_ref[0], w1tb_ref[...], ((0,), (0,)))
          + b1_ref[...])
    y1_ref[0] = y1
    s1_ref[0, 0, 0:1, :] = jnp.sum(y1, axis=0, keepdims=True)
    s1_ref[0, 0, 1:2, :] = jnp.sum(y1 * y1, axis=0, keepdims=True)


def _bn_relu(y, stats, g_ref, be_ref, count):
    ssum = jnp.sum(stats[:, :, 0, :], axis=(0, 1), keepdims=False)
    ssq = jnp.sum(stats[:, :, 1, :], axis=(0, 1), keepdims=False)
    mu = (ssum / count)[None, :]
    var = (ssq / count)[None, :] - mu * mu
    rstd = jax.lax.rsqrt(var + 1e-5)
    z = (y - mu) * rstd * g_ref[...] + be_ref[...]
    return jnp.maximum(z, 0.0)


def _k2_body(y1_ref, s1_ref, g1_ref, be1_ref, w2t_ref, b2_ref, y2_ref, s2_ref,
             *, count):
    z = _bn_relu(y1_ref[0], s1_ref[...], g1_ref, be1_ref, count)
    y2 = jnp.dot(z, w2t_ref[...], preferred_element_type=jnp.float32) + b2_ref[...]
    y2_ref[0] = y2
    s2_ref[0, 0, 0:1, :] = jnp.sum(y2, axis=0, keepdims=True)
    s2_ref[0, 0, 1:2, :] = jnp.sum(y2 * y2, axis=0, keepdims=True)


def _k3_body(y2_ref, s2_ref, g2_ref, be2_ref, out_ref, *, count):
    z = _bn_relu(y2_ref[0], s2_ref[...], g2_ref, be2_ref, count)
    out_ref[0] = z.T


def kernel(unknown_coords, known_coords, unknown_feats, known_feats,
           W1, b1, g1, be1, W2, b2, g2, be2):
    B, _, n = unknown_coords.shape
    m = known_coords.shape[2]
    C1 = unknown_feats.shape[1]
    C2 = known_feats.shape[1]
    CO1 = W1.shape[0]
    CO2 = W2.shape[0]
    nb = n // NBLK
    count = float(B * n)

    w1ta = W1[:, :C2].T
    w1tb = W1[:, C2:].T
    w2t = W2.T
    b1r = b1.reshape(1, CO1)
    g1r = g1.reshape(1, CO1)
    be1r = be1.reshape(1, CO1)
    b2r = b2.reshape(1, CO2)
    g2r = g2.reshape(1, CO2)
    be2r = be2.reshape(1, CO2)

    NB2 = min(2048, n)
    nb2 = n // NB2
    grid = (B, nb)
    grid2 = (B, nb2)
    f32 = jnp.float32

    y1, s1 = pl.pallas_call(
        _k1_body,
        grid=grid,
        in_specs=[
            pl.BlockSpec((1, 3, NBLK), lambda b, i: (b, 0, i)),
            pl.BlockSpec((1, 3, m), lambda b, i: (b, 0, 0)),
            pl.BlockSpec((1, C1, NBLK), lambda b, i: (b, 0, i)),
            pl.BlockSpec((1, C2, m), lambda b, i: (b, 0, 0)),
            pl.BlockSpec((C2, CO1), lambda b, i: (0, 0)),
            pl.BlockSpec((C1, CO1), lambda b, i: (0, 0)),
            pl.BlockSpec((1, CO1), lambda b, i: (0, 0)),
        ],
        out_specs=[
            pl.BlockSpec((1, NBLK, CO1), lambda b, i: (b, i, 0)),
            pl.BlockSpec((1, 1, 2, CO1), lambda b, i: (b, i, 0, 0)),
        ],
        out_shape=[
            jax.ShapeDtypeStruct((B, n, CO1), f32),
            jax.ShapeDtypeStruct((B, nb, 2, CO1), f32),
        ],
    )(unknown_coords, -2.0 * known_coords, unknown_feats, known_feats,
      w1ta, w1tb, b1r)

    from functools import partial
    y2, s2 = pl.pallas_call(
        partial(_k2_body, count=count),
        grid=grid2,
        in_specs=[
            pl.BlockSpec((1, NB2, CO1), lambda b, i: (b, i, 0)),
            pl.BlockSpec((B, nb, 2, CO1), lambda b, i: (0, 0, 0, 0)),
            pl.BlockSpec((1, CO1), lambda b, i: (0, 0)),
            pl.BlockSpec((1, CO1), lambda b, i: (0, 0)),
            pl.BlockSpec((CO1, CO2), lambda b, i: (0, 0)),
            pl.BlockSpec((1, CO2), lambda b, i: (0, 0)),
        ],
        out_specs=[
            pl.BlockSpec((1, NB2, CO2), lambda b, i: (b, i, 0)),
            pl.BlockSpec((1, 1, 2, CO2), lambda b, i: (b, i, 0, 0)),
        ],
        out_shape=[
            jax.ShapeDtypeStruct((B, n, CO2), f32),
            jax.ShapeDtypeStruct((B, nb2, 2, CO2), f32),
        ],
    )(y1, s1, g1r, be1r, w2t, b2r)

    out = pl.pallas_call(
        partial(_k3_body, count=count),
        grid=grid2,
        in_specs=[
            pl.BlockSpec((1, NB2, CO2), lambda b, i: (b, i, 0)),
            pl.BlockSpec((B, nb2, 2, CO2), lambda b, i: (0, 0, 0, 0)),
            pl.BlockSpec((1, CO2), lambda b, i: (0, 0)),
            pl.BlockSpec((1, CO2), lambda b, i: (0, 0)),
        ],
        out_specs=pl.BlockSpec((1, CO2, NB2), lambda b, i: (b, 0, i)),
        out_shape=jax.ShapeDtypeStruct((B, CO2, n), f32),
    )(y2, s2, g2r, be2r)
    return out


# speedup vs baseline: 2.2987x; 1.0349x over previous
import jax
import jax.numpy as jnp
from jax.experimental import pallas as pl

KNN = 3
NBLK = 1024


def _dotg(a, b, dims):
    return jax.lax.dot_general(a, b, (dims, ((), ())),
                               preferred_element_type=jnp.float32)


def _k1_body(uc_ref, kcs_ref, uf_ref, kf_ref, w1ta_ref, w1tb_ref, b1_ref,
             y1_ref, s1_ref):
    uc = uc_ref[0]
    kcs = kcs_ref[0]
    n, m = uc.shape[1], kcs.shape[1]

    su = jnp.sum(uc * uc, axis=0, keepdims=True).T
    sv = 0.25 * jnp.sum(kcs * kcs, axis=0, keepdims=True)
    ndot2 = _dotg(uc, kcs, ((0,), (0,)))
    d = (su + sv) + ndot2

    v1 = jnp.min(d, axis=1, keepdims=True)
    d2 = jnp.where(d == v1, jnp.inf, d)
    v2 = jnp.min(d2, axis=1, keepdims=True)
    d3 = jnp.where(d2 == v2, jnp.inf, d2)
    v3 = jnp.min(d3, axis=1, keepdims=True)
    dists = [v1, v2, v3]

    wk = [1.0 / (jnp.maximum(v, 0.0) + 1e-8) for v in dists]
    ws = (wk[0] + wk[1]) + wk[2]
    wn = [w / ws for w in wk]

    s_mat = jnp.where(d == v1, wn[0],
                      jnp.where(d == v2, wn[1],
                                jnp.where(d3 == v3, wn[2], 0.0)))
    inter = _dotg(s_mat, kf_ref[0], ((1,), (1,)))

    y1 = (jnp.dot(inter, w1ta_ref[...], preferred_element_type=jnp.float32)
          + _dotg(uf_ref[0], w1tb_ref[...], ((0,), (0,)))
          + b1_ref[...])
    y1_ref[0] = y1
    s1_ref[0, 0, 0:1, :] = jnp.sum(y1, axis=0, keepdims=True)
    s1_ref[0, 0, 1:2, :] = jnp.sum(y1 * y1, axis=0, keepdims=True)


def _bn_relu(y, stats, g_ref, be_ref, count):
    ssum = jnp.sum(stats[:, :, 0, :], axis=(0, 1), keepdims=False)
    ssq = jnp.sum(stats[:, :, 1, :], axis=(0, 1), keepdims=False)
    mu = (ssum / count)[None, :]
    var = (ssq / count)[None, :] - mu * mu
    rstd = jax.lax.rsqrt(var + 1e-5)
    z = (y - mu) * rstd * g_ref[...] + be_ref[...]
    return jnp.maximum(z, 0.0)


def _k2_body(y1_ref, s1_ref, g1_ref, be1_ref, w2t_ref, b2_ref, y2_ref, s2_ref,
             *, count):
    z = _bn_relu(y1_ref[0], s1_ref[...], g1_ref, be1_ref, count)
    y2 = jnp.dot(z, w2t_ref[...], preferred_element_type=jnp.float32) + b2_ref[...]
    y2_ref[0] = y2
    s2_ref[0, 0, 0:1, :] = jnp.sum(y2, axis=0, keepdims=True)
    s2_ref[0, 0, 1:2, :] = jnp.sum(y2 * y2, axis=0, keepdims=True)


def _k3_body(y2_ref, s2_ref, g2_ref, be2_ref, out_ref, *, count):
    z = _bn_relu(y2_ref[0], s2_ref[...], g2_ref, be2_ref, count)
    out_ref[0] = z.T


def kernel(unknown_coords, known_coords, unknown_feats, known_feats,
           W1, b1, g1, be1, W2, b2, g2, be2):
    B, _, n = unknown_coords.shape
    m = known_coords.shape[2]
    C1 = unknown_feats.shape[1]
    C2 = known_feats.shape[1]
    CO1 = W1.shape[0]
    CO2 = W2.shape[0]
    nb = n // NBLK
    count = float(B * n)

    w1ta = W1[:, :C2].T
    w1tb = W1[:, C2:].T
    w2t = W2.T
    b1r = b1.reshape(1, CO1)
    g1r = g1.reshape(1, CO1)
    be1r = be1.reshape(1, CO1)
    b2r = b2.reshape(1, CO2)
    g2r = g2.reshape(1, CO2)
    be2r = be2.reshape(1, CO2)

    NB2 = min(8192, n)
    nb2 = n // NB2
    grid = (B, nb)
    grid2 = (B, nb2)
    f32 = jnp.float32

    y1, s1 = pl.pallas_call(
        _k1_body,
        grid=grid,
        in_specs=[
            pl.BlockSpec((1, 3, NBLK), lambda b, i: (b, 0, i)),
            pl.BlockSpec((1, 3, m), lambda b, i: (b, 0, 0)),
            pl.BlockSpec((1, C1, NBLK), lambda b, i: (b, 0, i)),
            pl.BlockSpec((1, C2, m), lambda b, i: (b, 0, 0)),
            pl.BlockSpec((C2, CO1), lambda b, i: (0, 0)),
            pl.BlockSpec((C1, CO1), lambda b, i: (0, 0)),
            pl.BlockSpec((1, CO1), lambda b, i: (0, 0)),
        ],
        out_specs=[
            pl.BlockSpec((1, NBLK, CO1), lambda b, i: (b, i, 0)),
            pl.BlockSpec((1, 1, 2, CO1), lambda b, i: (b, i, 0, 0)),
        ],
        out_shape=[
            jax.ShapeDtypeStruct((B, n, CO1), f32),
            jax.ShapeDtypeStruct((B, nb, 2, CO1), f32),
        ],
    )(unknown_coords, -2.0 * known_coords, unknown_feats, known_feats,
      w1ta, w1tb, b1r)

    from functools import partial
    y2, s2 = pl.pallas_call(
        partial(_k2_body, count=count),
        grid=grid2,
        in_specs=[
            pl.BlockSpec((1, NB2, CO1), lambda b, i: (b, i, 0)),
            pl.BlockSpec((B, nb, 2, CO1), lambda b, i: (0, 0, 0, 0)),
            pl.BlockSpec((1, CO1), lambda b, i: (0, 0)),
            pl.BlockSpec((1, CO1), lambda b, i: (0, 0)),
            pl.BlockSpec((CO1, CO2), lambda b, i: (0, 0)),
            pl.BlockSpec((1, CO2), lambda b, i: (0, 0)),
        ],
        out_specs=[
            pl.BlockSpec((1, NB2, CO2), lambda b, i: (b, i, 0)),
            pl.BlockSpec((1, 1, 2, CO2), lambda b, i: (b, i, 0, 0)),
        ],
        out_shape=[
            jax.ShapeDtypeStruct((B, n, CO2), f32),
            jax.ShapeDtypeStruct((B, nb2, 2, CO2), f32),
        ],
    )(y1, s1, g1r, be1r, w2t, b2r)

    out = pl.pallas_call(
        partial(_k3_body, count=count),
        grid=grid2,
        in_specs=[
            pl.BlockSpec((1, NB2, CO2), lambda b, i: (b, i, 0)),
            pl.BlockSpec((B, nb2, 2, CO2), lambda b, i: (0, 0, 0, 0)),
            pl.BlockSpec((1, CO2), lambda b, i: (0, 0)),
            pl.BlockSpec((1, CO2), lambda b, i: (0, 0)),
        ],
        out_specs=pl.BlockSpec((1, CO2, NB2), lambda b, i: (b, 0, i)),
        out_shape=jax.ShapeDtypeStruct((B, CO2, n), f32),
    )(y2, s2, g2r, be2r)
    return out
